# baseline (speedup 1.0000x reference)
import functools

import jax
import jax.numpy as jnp
from jax import lax
from jax.experimental import pallas as pl
from jax.experimental.pallas import tpu as pltpu
from jax.experimental.pallas import tpu_sc as plsc

_NC = 2
_NS = 16
_LANES = 16
_BLK = 128


def _edge_proj(edge_attr, We_l):
    E, DE = edge_attr.shape
    H = We_l.shape[1]
    BE = 2560
    assert E % BE == 0

    def body(a_ref, w_ref, o_ref):
        o_ref[...] = lax.dot_general(
            a_ref[...], w_ref[...], (((1,), (0,)), ((), ())),
            preferred_element_type=jnp.float32)

    edge_attr = edge_attr.astype(jnp.bfloat16)
    We_l = We_l.astype(jnp.bfloat16)

    return pl.pallas_call(
        body,
        grid=(E // BE,),
        in_specs=[pl.BlockSpec((BE, DE), lambda i: (i, 0)),
                  pl.BlockSpec((DE, H), lambda i: (0, 0))],
        out_specs=pl.BlockSpec((BE, H), lambda i: (i, 0)),
        out_shape=jax.ShapeDtypeStruct((E, H), jnp.float32),
    )(edge_attr, We_l)


@functools.cache
def _make_edge_agg(N, E, H):
    NW = _NC * _NS
    BLK = 64
    n_blocks = E // BLK
    assert n_blocks * BLK == E
    bpt = (n_blocks + NW - 1) // NW
    zfull = N // BLK
    zrem = N - zfull * BLK
    zch_per_tile = (zfull + _NS) // _NS
    mesh = plsc.VectorSubcoreMesh(core_axis_name="c", subcore_axis_name="s")

    @functools.partial(
        pl.kernel,
        mesh=mesh,
        out_type=jax.ShapeDtypeStruct((_NC, N, H), jnp.float32),
        scratch_types=[
            pltpu.VMEM((2, BLK), jnp.int32),
            pltpu.VMEM((3, BLK), jnp.int32),
            pltpu.VMEM((2, BLK, H), jnp.float32),
            pltpu.VMEM((3, BLK, H), jnp.float32),
            pltpu.VMEM_SHARED((N, H), jnp.float32),
            pltpu.SemaphoreType.DMA,
            pltpu.SemaphoreType.DMA,
            pltpu.SemaphoreType.DMA,
            pltpu.SemaphoreType.DMA,
            pltpu.SemaphoreType.DMA,
            pltpu.SemaphoreType.DMA,
            pltpu.SemaphoreType.DMA,
            pltpu.SemaphoreType.DMA,
            pltpu.SemaphoreType.DMA,
            pltpu.SemaphoreType.DMA,
            pltpu.SemaphoreType.DMA,
            pltpu.SemaphoreType.DMA,
            pltpu.SemaphoreType.DMA,
        ],
    )
    def edge_agg(h_hbm, e_hbm, src_hbm, dst_hbm, out_hbm,
                 src2, dst3, hrows2, erows3, agg_sh,
                 si0, si1, sd0, sd1, sd2, sg0, sg1, se0, se1, se2,
                 ss0, ss1, ss2):
        c = lax.axis_index("c")
        s = lax.axis_index("s")
        wid = c * _NS + s
        sem_si = (si0, si1)
        sem_di = (sd0, sd1, sd2)
        sem_g = (sg0, sg1)
        sem_e = (se0, se1, se2)
        sem_s = (ss0, ss1, ss2)

        zvec = jnp.zeros((_LANES,), jnp.float32)

        @pl.loop(0, BLK)
        def _(i):
            for j in range(H // _LANES):
                hrows2[0, i, pl.ds(j * _LANES, _LANES)] = zvec

        @pl.loop(0, zch_per_tile)
        def _(k):
            ch = k * _NS + s

            @pl.when(ch < zfull)
            def _():
                pltpu.sync_copy(hrows2.at[0], agg_sh.at[pl.ds(ch * BLK, BLK)])

            if zrem:
                @pl.when(ch == zfull)
                def _():
                    pltpu.sync_copy(hrows2.at[0].at[pl.ds(0, zrem)],
                                    agg_sh.at[pl.ds(zfull * BLK, zrem)])

        plsc.subcore_barrier()

        def gid(i):
            return i * NW + wid

        def ok(i):
            return gid(i) < n_blocks

        def start_idx(i, hs, es):
            base = gid(i) * BLK
            pltpu.async_copy(src_hbm.at[pl.ds(base, BLK)], src2.at[hs],
                             sem_si[hs])
            pltpu.async_copy(dst_hbm.at[pl.ds(base, BLK)], dst3.at[es],
                             sem_di[es])

        def wait_idx(hs, es):
            pltpu.make_async_copy(src_hbm.at[pl.ds(0, BLK)], src2.at[hs],
                                  sem_si[hs]).wait()
            pltpu.make_async_copy(dst_hbm.at[pl.ds(0, BLK)], dst3.at[es],
                                  sem_di[es]).wait()

        def start_data(i, hs, es):
            base = gid(i) * BLK
            pltpu.async_copy(h_hbm.at[src2.at[hs]], hrows2.at[hs],
                             sem_g[hs])
            pltpu.async_copy(e_hbm.at[pl.ds(base, BLK)], erows3.at[es],
                             sem_e[es])

        def wait_data(hs, es):
            pltpu.make_async_copy(h_hbm.at[src2.at[hs]], hrows2.at[hs],
                                  sem_g[hs]).wait()
            pltpu.make_async_copy(e_hbm.at[pl.ds(0, BLK)], erows3.at[es],
                                  sem_e[es]).wait()

        def start_scatter(es):
            pltpu.async_copy(erows3.at[es], agg_sh.at[dst3.at[es]],
                             sem_s[es], add=True)

        def wait_scatter(es):
            pltpu.make_async_copy(erows3.at[es], agg_sh.at[dst3.at[es]],
                                  sem_s[es]).wait()

        def step(i, u, first=False):
            hs = u % 2
            hn = (u + 1) % 2
            es = u % 3
            en = (u + 1) % 3
            ep = (u + 2) % 3

            @pl.when(ok(i))
            def _():
                wait_data(hs, es)

            @pl.when(ok(i + 1))
            def _():
                wait_idx(hn, en)
                start_data(i + 1, hn, en)

            @pl.when(ok(i))
            def _():
                @pl.loop(0, BLK)
                def _(k):
                    for j in range(H // _LANES):
                        sl = pl.ds(j * _LANES, _LANES)
                        hv = hrows2[hs, k, sl]
                        ev = erows3[es, k, sl]
                        erows3[es, k, sl] = jnp.maximum(hv + ev, 0.0)

                if not first:
                    wait_scatter(ep)

                start_scatter(es)

                @pl.when(ok(i + 2))
                def _():
                    start_idx(i + 2, hs, ep)

        start_idx(0, 0, 0)
        wait_idx(0, 0)
        start_data(0, 0, 0)
        start_idx(1, 1, 1)

        step(0, 0, first=True)

        n_rest = bpt - 1
        assert n_rest % 6 == 0, n_rest

        @pl.loop(0, n_rest // 6)
        def _(k):
            i0 = k * 6 + 1
            for t in range(6):
                step(i0 + t, (t + 1) % 6)

        @pl.when(ok(bpt - 1))
        def _():
            wait_scatter((bpt - 1) % 3)

        @pl.when(jnp.logical_and(jnp.logical_not(ok(bpt - 1)), ok(bpt - 2)))
        def _():
            wait_scatter((bpt - 2) % 3)

        plsc.subcore_barrier()

        @pl.loop(0, zch_per_tile)
        def _(k):
            ch = k * _NS + s

            @pl.when(ch < zfull)
            def _():
                pltpu.sync_copy(agg_sh.at[pl.ds(ch * BLK, BLK)],
                                out_hbm.at[c].at[pl.ds(ch * BLK, BLK)])

            if zrem:
                @pl.when(ch == zfull)
                def _():
                    pltpu.sync_copy(agg_sh.at[pl.ds(zfull * BLK, zrem)],
                                    out_hbm.at[c].at[pl.ds(zfull * BLK, zrem)])

    return edge_agg


def _node_update(h_in, parts, W1l, b1l, W2l, b2l, gammal, betal, relu_out):
    N, H = h_in.shape

    def body(h_ref, p_ref, w1, b1, w2, b2, ga, be, o_ref):
        z = h_ref[...] + p_ref[0] + p_ref[1]
        u = lax.dot_general(z, w1[...], (((1,), (0,)), ((), ())),
                            preferred_element_type=jnp.float32) + b1[...]
        u = jnp.maximum(u, 0.0)
        v = lax.dot_general(u, w2[...], (((1,), (0,)), ((), ())),
                            preferred_element_type=jnp.float32) + b2[...]
        mu = jnp.mean(v, axis=0, keepdims=True)
        var = jnp.mean((v - mu) * (v - mu), axis=0, keepdims=True)
        zn = (v - mu) * lax.rsqrt(var + 1e-5) * ga[...] + be[...]
        if relu_out:
            zn = jnp.maximum(zn, 0.0)
        o_ref[...] = zn + h_ref[...]

    return pl.pallas_call(
        body,
        out_shape=jax.ShapeDtypeStruct((N, H), jnp.float32),
    )(h_in, parts, W1l, b1l, W2l, b2l, gammal, betal)


def _readout(h, batch_col, G):
    N, H = h.shape

    GB = 8

    def body(h_ref, b_ref, o_ref):
        g0 = pl.program_id(0) * GB
        hv = h_ref[...]
        bv = b_ref[...]
        rows = [jnp.max(jnp.where(bv == g0 + gg, hv, -jnp.inf),
                        axis=0, keepdims=True)
                for gg in range(GB)]
        o_ref[...] = jnp.concatenate(rows, axis=0)

    return pl.pallas_call(
        body,
        grid=(G // GB,),
        in_specs=[pl.BlockSpec((N, H), lambda g: (0, 0)),
                  pl.BlockSpec((N, 1), lambda g: (0, 0))],
        out_specs=pl.BlockSpec((GB, H), lambda g: (g, 0)),
        out_shape=jax.ShapeDtypeStruct((G, H), jnp.float32),
    )(h, batch_col)


def kernel(x, edge_index, edge_attr, batch, W1, b1, W2, b2, We, gamma, beta):
    N, H = x.shape
    E = edge_index.shape[1]
    L = W1.shape[0]
    G = 128

    src = edge_index[0]
    dst = edge_index[1]
    edge_agg = _make_edge_agg(N, E, H)

    es = [_edge_proj(edge_attr, We[l]) for l in range(L)]
    h = x
    for l in range(L):
        parts = edge_agg(h, es[l], src, dst)
        h = _node_update(h, parts,
                         W1[l], b1[l].reshape(1, -1),
                         W2[l], b2[l].reshape(1, -1),
                         gamma[l].reshape(1, -1), beta[l].reshape(1, -1),
                         relu_out=(l < L - 1))
    h_rep = _readout(h, batch.reshape(-1, 1), G)
    return h_rep, h


# speedup vs baseline: 1.1467x; 1.1467x over previous
import dataclasses
import functools

import jax
import jax.numpy as jnp
from jax import lax
from jax.experimental import pallas as pl
from jax.experimental.pallas import tpu as pltpu
from jax.experimental.pallas import tpu_sc as plsc

_NC = 2
_NS = 16
_LANES = 16
_BLK = 128


def _edge_proj(edge_attr, We_l):
    E, DE = edge_attr.shape
    H = We_l.shape[1]
    BE = 2560
    assert E % BE == 0

    def body(a_ref, w_ref, o_ref):
        o_ref[...] = lax.dot_general(
            a_ref[...], w_ref[...], (((1,), (0,)), ((), ())),
            preferred_element_type=jnp.float32)

    edge_attr = edge_attr.astype(jnp.bfloat16)
    We_l = We_l.astype(jnp.bfloat16)

    return pl.pallas_call(
        body,
        grid=(E // BE,),
        in_specs=[pl.BlockSpec((BE, DE), lambda i: (i, 0)),
                  pl.BlockSpec((DE, H), lambda i: (0, 0))],
        out_specs=pl.BlockSpec((BE, H), lambda i: (i, 0)),
        out_shape=jax.ShapeDtypeStruct((E, H), jnp.float32),
    )(edge_attr, We_l)


@functools.cache
def _make_edge_agg(N, E, H):
    NW = _NC * _NS
    BLK = 80
    n_blocks = E // BLK
    bpt = n_blocks // NW
    assert n_blocks == bpt * NW
    zch = N // BLK
    assert zch * BLK == N
    zch_per_tile = (zch + _NS - 1) // _NS
    mesh = plsc.VectorSubcoreMesh(core_axis_name="c", subcore_axis_name="s")

    @functools.partial(
        pl.kernel,
        mesh=mesh,
        out_type=jax.ShapeDtypeStruct((_NC, N, H), jnp.float32),
        scratch_types=[
            pltpu.VMEM((2, BLK), jnp.int32),
            pltpu.VMEM((2, BLK), jnp.int32),
            pltpu.VMEM((2, BLK, H), jnp.float32),
            pltpu.VMEM((2, BLK, H), jnp.float32),
            pltpu.VMEM_SHARED((N, H), jnp.float32),
            pltpu.SemaphoreType.DMA,
            pltpu.SemaphoreType.DMA,
            pltpu.SemaphoreType.DMA,
            pltpu.SemaphoreType.DMA,
            pltpu.SemaphoreType.DMA,
            pltpu.SemaphoreType.DMA,
        ],
    )
    def edge_agg(h_hbm, e_hbm, src_hbm, dst_hbm, out_hbm,
                 src2, dst2, hrows2, erows2, agg_sh,
                 sem_i0, sem_i1, sem_g0, sem_g1, sem_e0, sem_e1):
        c = lax.axis_index("c")
        s = lax.axis_index("s")
        wid = c * _NS + s
        blk0 = wid * bpt
        sem_i = (sem_i0, sem_i1)
        sem_g = (sem_g0, sem_g1)
        sem_e = (sem_e0, sem_e1)

        zvec = jnp.zeros((_LANES,), jnp.float32)

        @pl.loop(0, BLK)
        def _(i):
            for j in range(H // _LANES):
                hrows2[0, i, pl.ds(j * _LANES, _LANES)] = zvec

        @pl.loop(0, zch_per_tile)
        def _(k):
            ch = k * _NS + s

            @pl.when(ch < zch)
            def _():
                pltpu.sync_copy(hrows2.at[0], agg_sh.at[pl.ds(ch * BLK, BLK)])

        plsc.subcore_barrier()

        def start_idx(slot, i):
            base = (blk0 + i) * BLK
            pltpu.async_copy(src_hbm.at[pl.ds(base, BLK)], src2.at[slot],
                             sem_i[slot])
            pltpu.async_copy(dst_hbm.at[pl.ds(base, BLK)], dst2.at[slot],
                             sem_i[slot])

        def wait_idx(slot):
            pltpu.make_async_copy(src_hbm.at[pl.ds(0, BLK)], src2.at[slot],
                                  sem_i[slot]).wait()
            pltpu.make_async_copy(dst_hbm.at[pl.ds(0, BLK)], dst2.at[slot],
                                  sem_i[slot]).wait()

        def start_data(slot, i):
            base = (blk0 + i) * BLK
            pltpu.async_copy(h_hbm.at[src2.at[slot]], hrows2.at[slot],
                             sem_g[slot])
            pltpu.async_copy(e_hbm.at[pl.ds(base, BLK)], erows2.at[slot],
                             sem_e[slot])

        def wait_data(slot):
            pltpu.make_async_copy(h_hbm.at[src2.at[slot]], hrows2.at[slot],
                                  sem_g[slot]).wait()
            pltpu.make_async_copy(e_hbm.at[pl.ds(0, BLK)], erows2.at[slot],
                                  sem_e[slot]).wait()

        def step(i, slot, nxt):
            wait_data(slot)

            @pl.when(i + 1 < bpt)
            def _():
                wait_idx(nxt)
                start_data(nxt, i + 1)

            @pl.loop(0, BLK)
            def _(k):
                for j in range(H // _LANES):
                    sl = pl.ds(j * _LANES, _LANES)
                    hv = hrows2[slot, k, sl]
                    ev = erows2[slot, k, sl]
                    hrows2[slot, k, sl] = jnp.maximum(hv + ev, 0.0)

            pltpu.sync_copy(hrows2.at[slot], agg_sh.at[dst2.at[slot]],
                            add=True)

            @pl.when(i + 2 < bpt)
            def _():
                start_idx(slot, i + 2)

        start_idx(0, 0)
        wait_idx(0)
        start_data(0, 0)
        start_idx(1, 1)

        @pl.loop(0, bpt // 2)
        def _(k):
            i = k * 2
            step(i, 0, 1)
            step(i + 1, 1, 0)

        if bpt % 2:
            step(bpt - 1, 0, 1)

        plsc.subcore_barrier()

        @pl.loop(0, zch_per_tile)
        def _(k):
            ch = k * _NS + s

            @pl.when(ch < zch)
            def _():
                pltpu.sync_copy(agg_sh.at[pl.ds(ch * BLK, BLK)],
                                out_hbm.at[c].at[pl.ds(ch * BLK, BLK)])

    return edge_agg


def _node_update(h_in, parts, W1l, b1l, W2l, b2l, gammal, betal, relu_out):
    N, H = h_in.shape

    def body(h_ref, p_ref, w1, b1, w2, b2, ga, be, o_ref):
        z = h_ref[...] + p_ref[0] + p_ref[1]
        u = lax.dot_general(z, w1[...], (((1,), (0,)), ((), ())),
                            preferred_element_type=jnp.float32) + b1[...]
        u = jnp.maximum(u, 0.0)
        v = lax.dot_general(u, w2[...], (((1,), (0,)), ((), ())),
                            preferred_element_type=jnp.float32) + b2[...]
        mu = jnp.mean(v, axis=0, keepdims=True)
        var = jnp.mean((v - mu) * (v - mu), axis=0, keepdims=True)
        zn = (v - mu) * lax.rsqrt(var + 1e-5) * ga[...] + be[...]
        if relu_out:
            zn = jnp.maximum(zn, 0.0)
        o_ref[...] = zn + h_ref[...]

    return pl.pallas_call(
        body,
        out_shape=jax.ShapeDtypeStruct((N, H), jnp.float32),
    )(h_in, parts, W1l, b1l, W2l, b2l, gammal, betal)


@functools.cache
def _make_readout(N, H, G):
    NW = _NC * _NS
    GPT = G // NW
    CH = N // _LANES
    assert CH * _LANES == N
    W = 64
    mesh = plsc.VectorSubcoreMesh(core_axis_name="c", subcore_axis_name="s")
    cp = pltpu.CompilerParams()
    if "needs_layout_passes" in pltpu.CompilerParams.__dataclass_fields__:
        cp = dataclasses.replace(cp, needs_layout_passes=False)

    @functools.partial(
        pl.kernel,
        mesh=mesh,
        compiler_params=cp,
        out_type=jax.ShapeDtypeStruct((NW, GPT, H), jnp.float32),
        scratch_types=[
            pltpu.VMEM((N,), jnp.int32),
            pltpu.VMEM((W, H), jnp.float32),
            pltpu.VMEM((GPT, H), jnp.float32),
            pltpu.SemaphoreType.DMA,
        ],
    )
    def readout(h_hbm, b_hbm, out_hbm, bv, rows, acc, sem):
        c = lax.axis_index("c")
        s = lax.axis_index("s")
        wid = c * _NS + s
        g0 = wid * GPT

        pltpu.async_copy(b_hbm, bv, sem).wait()

        zero = jnp.zeros((_LANES,), jnp.int32)
        one = jnp.ones((_LANES,), jnp.int32)

        @pl.loop(0, CH, init_carry=(zero,) * (GPT + 1))
        def counts(k, carry):
            ch = bv[pl.ds(k * _LANES, _LANES)]
            return tuple(
                carry[q] + jnp.where(ch < g0 + q, one, zero)
                for q in range(GPT + 1))

        cnts = [jnp.sum(v) for v in counts]

        ninf = jnp.full((_LANES,), -jnp.inf, jnp.float32)
        for q in range(GPT):
            for j in range(H // _LANES):
                acc[q, pl.ds(j * _LANES, _LANES)] = ninf

        for q in range(GPT):
            start = cnts[q]
            end = cnts[q + 1]
            start8 = start - lax.rem(start, 8)
            nwin = jnp.maximum((end - start8 + W - 1) // W, 0)

            @pl.loop(0, nwin)
            def _(k):
                w0 = pl.multiple_of(jnp.minimum(start8 + k * W, N - W), 8)
                pltpu.async_copy(h_hbm.at[pl.ds(w0, W)], rows, sem).wait()

                @pl.loop(0, W)
                def _(r):
                    row = w0 + r

                    @pl.when(jnp.logical_and(row >= start, row < end))
                    def _():
                        for j in range(H // _LANES):
                            sl = pl.ds(j * _LANES, _LANES)
                            acc[q, sl] = jnp.maximum(acc[q, sl], rows[r, sl])

        pltpu.sync_copy(acc, out_hbm.at[wid])

    return readout


def kernel(x, edge_index, edge_attr, batch, W1, b1, W2, b2, We, gamma, beta):
    N, H = x.shape
    E = edge_index.shape[1]
    L = W1.shape[0]
    G = 128

    src = edge_index[0]
    dst = edge_index[1]
    edge_agg = _make_edge_agg(N, E, H)

    es = [_edge_proj(edge_attr, We[l]) for l in range(L)]
    h = x
    for l in range(L):
        parts = edge_agg(h, es[l], src, dst)
        h = _node_update(h, parts,
                         W1[l], b1[l].reshape(1, -1),
                         W2[l], b2[l].reshape(1, -1),
                         gamma[l].reshape(1, -1), beta[l].reshape(1, -1),
                         relu_out=(l < L - 1))
    h_rep = _make_readout(N, H, G)(h, batch).reshape(G, H)
    return h_rep, h


# speedup vs baseline: 1.1474x; 1.0006x over previous
import dataclasses
import functools

import jax
import jax.numpy as jnp
from jax import lax
from jax.experimental import pallas as pl
from jax.experimental.pallas import tpu as pltpu
from jax.experimental.pallas import tpu_sc as plsc

_NC = 2
_NS = 16
_LANES = 16
_BLK = 128


def _edge_proj(edge_attr, We_l):
    E, DE = edge_attr.shape
    H = We_l.shape[1]
    BE = 2560
    assert E % BE == 0

    def body(a_ref, w_ref, o_ref):
        o_ref[...] = lax.dot_general(
            a_ref[...], w_ref[...], (((1,), (0,)), ((), ())),
            preferred_element_type=jnp.float32)

    edge_attr = edge_attr.astype(jnp.bfloat16)
    We_l = We_l.astype(jnp.bfloat16)

    return pl.pallas_call(
        body,
        grid=(E // BE,),
        in_specs=[pl.BlockSpec((BE, DE), lambda i: (i, 0)),
                  pl.BlockSpec((DE, H), lambda i: (0, 0))],
        out_specs=pl.BlockSpec((BE, H), lambda i: (i, 0)),
        out_shape=jax.ShapeDtypeStruct((E, H), jnp.float32),
    )(edge_attr, We_l)


@functools.cache
def _make_edge_agg(N, E, H):
    NW = _NC * _NS
    BLK = 80
    n_blocks = E // BLK
    bpt = n_blocks // NW
    assert n_blocks == bpt * NW
    zch = N // BLK
    assert zch * BLK == N
    zch_per_tile = (zch + _NS - 1) // _NS
    mesh = plsc.VectorSubcoreMesh(core_axis_name="c", subcore_axis_name="s")

    @functools.partial(
        pl.kernel,
        mesh=mesh,
        out_type=jax.ShapeDtypeStruct((_NC, N, H), jnp.float32),
        scratch_types=[
            pltpu.VMEM((2, BLK), jnp.int32),
            pltpu.VMEM((2, BLK), jnp.int32),
            pltpu.VMEM((2, BLK, H), jnp.float32),
            pltpu.VMEM((2, BLK, H), jnp.float32),
            pltpu.VMEM_SHARED((N, H), jnp.float32),
            pltpu.SemaphoreType.DMA,
            pltpu.SemaphoreType.DMA,
            pltpu.SemaphoreType.DMA,
            pltpu.SemaphoreType.DMA,
            pltpu.SemaphoreType.DMA,
            pltpu.SemaphoreType.DMA,
        ],
    )
    def edge_agg(h_hbm, e_hbm, src_hbm, dst_hbm, out_hbm,
                 src2, dst2, hrows2, erows2, agg_sh,
                 sem_i0, sem_i1, sem_g0, sem_g1, sem_e0, sem_e1):
        c = lax.axis_index("c")
        s = lax.axis_index("s")
        wid = c * _NS + s
        blk0 = wid * bpt
        sem_i = (sem_i0, sem_i1)
        sem_g = (sem_g0, sem_g1)
        sem_e = (sem_e0, sem_e1)

        zvec = jnp.zeros((_LANES,), jnp.float32)

        @pl.loop(0, BLK)
        def _(i):
            for j in range(H // _LANES):
                hrows2[0, i, pl.ds(j * _LANES, _LANES)] = zvec

        @pl.loop(0, zch_per_tile)
        def _(k):
            ch = k * _NS + s

            @pl.when(ch < zch)
            def _():
                pltpu.sync_copy(hrows2.at[0], agg_sh.at[pl.ds(ch * BLK, BLK)])

        plsc.subcore_barrier()

        def start_idx(slot, i):
            base = (blk0 + i) * BLK
            pltpu.async_copy(src_hbm.at[pl.ds(base, BLK)], src2.at[slot],
                             sem_i[slot])
            pltpu.async_copy(dst_hbm.at[pl.ds(base, BLK)], dst2.at[slot],
                             sem_i[slot])

        def wait_idx(slot):
            pltpu.make_async_copy(src_hbm.at[pl.ds(0, BLK)], src2.at[slot],
                                  sem_i[slot]).wait()
            pltpu.make_async_copy(dst_hbm.at[pl.ds(0, BLK)], dst2.at[slot],
                                  sem_i[slot]).wait()

        def start_data(slot, i):
            base = (blk0 + i) * BLK
            pltpu.async_copy(h_hbm.at[src2.at[slot]], hrows2.at[slot],
                             sem_g[slot])
            pltpu.async_copy(e_hbm.at[pl.ds(base, BLK)], erows2.at[slot],
                             sem_e[slot])

        def wait_data(slot):
            pltpu.make_async_copy(h_hbm.at[src2.at[slot]], hrows2.at[slot],
                                  sem_g[slot]).wait()
            pltpu.make_async_copy(e_hbm.at[pl.ds(0, BLK)], erows2.at[slot],
                                  sem_e[slot]).wait()

        def step(i, slot, nxt):
            wait_data(slot)

            @pl.when(i + 1 < bpt)
            def _():
                wait_idx(nxt)
                start_data(nxt, i + 1)

            @pl.loop(0, BLK)
            def _(k):
                for j in range(H // _LANES):
                    sl = pl.ds(j * _LANES, _LANES)
                    hv = hrows2[slot, k, sl]
                    ev = erows2[slot, k, sl]
                    hrows2[slot, k, sl] = jnp.maximum(hv + ev, 0.0)

            pltpu.sync_copy(hrows2.at[slot], agg_sh.at[dst2.at[slot]],
                            add=True)

            @pl.when(i + 2 < bpt)
            def _():
                start_idx(slot, i + 2)

        start_idx(0, 0)
        wait_idx(0)
        start_data(0, 0)
        start_idx(1, 1)

        @pl.loop(0, bpt // 2)
        def _(k):
            i = k * 2
            step(i, 0, 1)
            step(i + 1, 1, 0)

        if bpt % 2:
            step(bpt - 1, 0, 1)

        plsc.subcore_barrier()

        @pl.loop(0, zch_per_tile)
        def _(k):
            ch = k * _NS + s

            @pl.when(ch < zch)
            def _():
                pltpu.sync_copy(agg_sh.at[pl.ds(ch * BLK, BLK)],
                                out_hbm.at[c].at[pl.ds(ch * BLK, BLK)])

    return edge_agg


def _node_update(h_in, parts, W1l, b1l, W2l, b2l, gammal, betal, relu_out):
    N, H = h_in.shape

    def body(h_ref, p_ref, w1, b1, w2, b2, ga, be, o_ref):
        z = h_ref[...] + p_ref[0] + p_ref[1]
        u = lax.dot_general(z.astype(jnp.bfloat16),
                            w1[...].astype(jnp.bfloat16),
                            (((1,), (0,)), ((), ())),
                            preferred_element_type=jnp.float32) + b1[...]
        u = jnp.maximum(u, 0.0)
        v = lax.dot_general(u.astype(jnp.bfloat16),
                            w2[...].astype(jnp.bfloat16),
                            (((1,), (0,)), ((), ())),
                            preferred_element_type=jnp.float32) + b2[...]
        mu = jnp.mean(v, axis=0, keepdims=True)
        var = jnp.mean((v - mu) * (v - mu), axis=0, keepdims=True)
        zn = (v - mu) * lax.rsqrt(var + 1e-5) * ga[...] + be[...]
        if relu_out:
            zn = jnp.maximum(zn, 0.0)
        o_ref[...] = zn + h_ref[...]

    return pl.pallas_call(
        body,
        out_shape=jax.ShapeDtypeStruct((N, H), jnp.float32),
    )(h_in, parts, W1l, b1l, W2l, b2l, gammal, betal)


@functools.cache
def _make_readout(N, H, G):
    NW = _NC * _NS
    GPT = G // NW
    CH = N // _LANES
    assert CH * _LANES == N
    W = 64
    mesh = plsc.VectorSubcoreMesh(core_axis_name="c", subcore_axis_name="s")
    cp = pltpu.CompilerParams()
    if "needs_layout_passes" in pltpu.CompilerParams.__dataclass_fields__:
        cp = dataclasses.replace(cp, needs_layout_passes=False)

    @functools.partial(
        pl.kernel,
        mesh=mesh,
        compiler_params=cp,
        out_type=jax.ShapeDtypeStruct((NW, GPT, H), jnp.float32),
        scratch_types=[
            pltpu.VMEM((N,), jnp.int32),
            pltpu.VMEM((W, H), jnp.float32),
            pltpu.VMEM((GPT, H), jnp.float32),
            pltpu.SemaphoreType.DMA,
        ],
    )
    def readout(h_hbm, b_hbm, out_hbm, bv, rows, acc, sem):
        c = lax.axis_index("c")
        s = lax.axis_index("s")
        wid = c * _NS + s
        g0 = wid * GPT

        pltpu.async_copy(b_hbm, bv, sem).wait()

        zero = jnp.zeros((_LANES,), jnp.int32)
        one = jnp.ones((_LANES,), jnp.int32)

        @pl.loop(0, CH, init_carry=(zero,) * (GPT + 1))
        def counts(k, carry):
            ch = bv[pl.ds(k * _LANES, _LANES)]
            return tuple(
                carry[q] + jnp.where(ch < g0 + q, one, zero)
                for q in range(GPT + 1))

        cnts = [jnp.sum(v) for v in counts]

        ninf = jnp.full((_LANES,), -jnp.inf, jnp.float32)
        for q in range(GPT):
            for j in range(H // _LANES):
                acc[q, pl.ds(j * _LANES, _LANES)] = ninf

        for q in range(GPT):
            start = cnts[q]
            end = cnts[q + 1]
            start8 = start - lax.rem(start, 8)
            nwin = jnp.maximum((end - start8 + W - 1) // W, 0)

            @pl.loop(0, nwin)
            def _(k):
                w0 = pl.multiple_of(jnp.minimum(start8 + k * W, N - W), 8)
                pltpu.async_copy(h_hbm.at[pl.ds(w0, W)], rows, sem).wait()

                @pl.loop(0, W)
                def _(r):
                    row = w0 + r

                    @pl.when(jnp.logical_and(row >= start, row < end))
                    def _():
                        for j in range(H // _LANES):
                            sl = pl.ds(j * _LANES, _LANES)
                            acc[q, sl] = jnp.maximum(acc[q, sl], rows[r, sl])

        pltpu.sync_copy(acc, out_hbm.at[wid])

    return readout


def kernel(x, edge_index, edge_attr, batch, W1, b1, W2, b2, We, gamma, beta):
    N, H = x.shape
    E = edge_index.shape[1]
    L = W1.shape[0]
    G = 128

    src = edge_index[0]
    dst = edge_index[1]
    edge_agg = _make_edge_agg(N, E, H)

    es = [_edge_proj(edge_attr, We[l]) for l in range(L)]
    h = x
    for l in range(L):
        parts = edge_agg(h, es[l], src, dst)
        h = _node_update(h, parts,
                         W1[l], b1[l].reshape(1, -1),
                         W2[l], b2[l].reshape(1, -1),
                         gamma[l].reshape(1, -1), beta[l].reshape(1, -1),
                         relu_out=(l < L - 1))
    h_rep = _make_readout(N, H, G)(h, batch).reshape(G, H)
    return h_rep, h
